# baseline (speedup 1.0000x reference)
import functools

import jax
import jax.numpy as jnp
from jax import lax
from jax.experimental import pallas as pl
from jax.experimental.pallas import tpu as pltpu
from jax.experimental.pallas import tpu_sc as plsc

_VOCAB = 1000000
_D = 16
_F = 26
_B = 16384

_SLAB = 512
_NSLAB_FULL = _VOCAB // _SLAB
_TAIL = _VOCAB - _NSLAB_FULL * _SLAB
_TRIPLES = 21

_NC = 2
_NS = 16
_NW = _NC * _NS
_ROWS_PER_W = _B // _NW
_CHUNK = 64
_NCHUNK = _ROWS_PER_W // _CHUNK
_IDX_PER_CHUNK = _CHUNK * _F
_GATHER = 128
_GPC = _IDX_PER_CHUNK // _GATHER
_IDXROWS_PER_W = _ROWS_PER_W * _F // _GATHER


def _tr_body(tab_t_hbm, out_hbm, slabs, out_bufs, tail_buf, sem_in, sem_out):
    wid = lax.axis_index("s") * _NC + lax.axis_index("c")
    nper = jnp.where(wid == 0, 62, 61)
    base = 61 * wid + jnp.minimum(wid, 1)

    lane = lax.iota(jnp.int32, 16)
    zq4 = lax.shift_right_logical(lane, 3) * 4
    yv = lax.bitwise_and(lane, 7)
    bufv = tuple(jnp.full((16,), b, jnp.int32) for b in range(3))

    def issue_in(s, buf):
        col0 = s * _SLAB
        for i in range(2):
            for jj in range(4):
                pltpu.async_copy(
                    tab_t_hbm.at[pl.ds(8 * i, 8),
                                 pl.ds(col0 + 128 * jj, 128)],
                    slabs.at[buf, i * 4 + jj], sem_in)

    def drain_in(buf):
        for t in range(8):
            pltpu.make_async_copy(
                tab_t_hbm.at[pl.ds(0, 8), pl.ds(0, 128)],
                slabs.at[buf, t], sem_in).wait()

    def drain_out(buf):
        pltpu.make_async_copy(out_hbm.at[pl.ds(0, _SLAB * 16)],
                              out_bufs.at[buf], sem_out).wait()

    diags = [lax.bitwise_and(lane + sh, 15) for sh in range(16)]
    sbases = [diags[sh] * 16 + lane for sh in range(16)]
    zero16 = jnp.zeros((16,), jnp.int32)
    gbase0 = zq4 * 1024 + yv * 128

    def compute(s, buf):
        col0 = s * _SLAB

        def tile_body(t, carry):
            gbt = gbase0 + (t * 1024 + buf * 8192)
            obase = t * 2048

            def g_body(g, carry2):
                cc0 = g * 16
                ga = gbt + cc0
                sb = lane + (obase + cc0 * 16)
                for sh in range(16):
                    v = plsc.load_gather(
                        slabs, [zero16, zero16, zero16, ga + diags[sh]])
                    plsc.store_scatter(
                        out_bufs, [bufv[buf], sb + sbases[sh]], v)
                return carry2

            lax.fori_loop(0, 8, g_body, carry)
            return carry

        lax.fori_loop(0, 4, tile_body, jnp.int32(0))
        pltpu.async_copy(out_bufs.at[buf],
                         out_hbm.at[pl.ds(col0 * 16, _SLAB * 16)], sem_out)

    issue_in(base, 0)
    issue_in(base + 1, 1)

    def slot(k, buf):
        @pl.when(k < nper)
        def _():
            drain_in(buf)

            @pl.when(k + 2 < nper)
            def _():
                issue_in(base + k + 2, (buf + 2) % 3)

            @pl.when(k >= 3)
            def _():
                drain_out(buf)

            compute(base + k, buf)

    def triple_body(p, carry):
        k = 3 * p
        slot(k, 0)
        slot(k + 1, 1)
        slot(k + 2, 2)
        return carry

    lax.fori_loop(0, _TRIPLES, triple_body, jnp.int32(0))
    drain_out(0)
    drain_out(1)
    drain_out(2)

    @pl.when(wid == _NW - 1)
    def _():
        tc0 = _NSLAB_FULL * _SLAB
        for d in range(_D):
            pltpu.sync_copy(tab_t_hbm.at[d, pl.ds(tc0, _TAIL)],
                            tail_buf.at[d])
        ga0 = lane * _TAIL
        for g in range(_TAIL // 16):
            cc0 = g * 16
            sb = lane + cc0 * 16
            for sh in range(16):
                v = plsc.load_gather(
                    tail_buf, [zero16, ga0 + (diags[sh] + cc0)])
                plsc.store_scatter(
                    out_bufs, [bufv[0], sb + sbases[sh]], v)
        pltpu.sync_copy(out_bufs.at[0, pl.ds(0, _TAIL * 16)],
                        out_hbm.at[pl.ds(tc0 * 16, _TAIL * 16)])


@jax.jit
def _tr_call(tab_t):
    mesh = plsc.VectorSubcoreMesh(core_axis_name="c", subcore_axis_name="s")
    return pl.kernel(
        _tr_body,
        mesh=mesh,
        out_type=jax.ShapeDtypeStruct((_VOCAB * _D,), jnp.float32),
        compiler_params=pltpu.CompilerParams(needs_layout_passes=False,
                                             use_tc_tiling_on_sc=True),
        scratch_types=[
            pltpu.VMEM((3, 8, 8, 128), jnp.float32),
            pltpu.VMEM((4, _SLAB * 16), jnp.float32),
            pltpu.VMEM((_D, _TAIL), jnp.float32),
            pltpu.SemaphoreType.DMA,
            pltpu.SemaphoreType.DMA,
        ],
    )(tab_t)


def _fm_body(idx_flat_hbm, table_hbm, wb_hbm, out_hbm,
             idx_lin, rows, out_v, wconst, sem0, sem1):
    wid = lax.axis_index("s") * _NC + lax.axis_index("c")
    sems = (sem0, sem1)

    pltpu.sync_copy(idx_flat_hbm.at[pl.ds(wid * (_ROWS_PER_W * _F),
                                          _ROWS_PER_W * _F)], idx_lin)
    pltpu.sync_copy(wb_hbm, wconst)

    w0 = wconst[pl.ds(0, 16)]
    w1 = wconst[pl.ds(16, 16)]
    bias16 = wconst[pl.ds(32, 16)]

    def issue(c, buf):
        handles = []
        for j in range(_GPC):
            g = c * _GPC + j
            handles.append(pltpu.async_copy(
                table_hbm.at[idx_lin.at[pl.ds(g * _GATHER, _GATHER)]],
                rows.at[buf, pl.ds(j * _GATHER, _GATHER), :],
                sems[buf]))
        return handles

    def compute(c, buf):
        def row_body(lb, _):
            base = lb * _F
            boff = c * _IDX_PER_CHUNK + base
            v = rows[buf, base, :]
            s = v
            q = v * v
            for f in range(1, _F):
                v = rows[buf, base + f, :]
                s = s + v
                q = q + v * v
            i0 = idx_lin[pl.ds(boff, 16)].astype(jnp.float32)
            i1 = idx_lin[pl.ds(boff + 10, 16)].astype(jnp.float32)
            t = 0.5 * (s * s - q) + i0 * w0 + i1 * w1 + bias16
            dest = jnp.full((16,), c * _CHUNK + lb, jnp.int32)
            plsc.addupdate_scatter(out_v, [dest], t)
            return _

        lax.fori_loop(0, _CHUNK, row_body, jnp.int32(0))

    def zero_out(k, _):
        out_v[pl.ds(k * 16, 16)] = jnp.zeros((16,), jnp.float32)
        return _

    lax.fori_loop(0, _ROWS_PER_W // 16, zero_out, jnp.int32(0))

    pending = issue(0, 0)
    for c in range(_NCHUNK):
        nxt = issue(c + 1, (c + 1) % 2) if c + 1 < _NCHUNK else []
        for h in pending:
            h.wait()
        compute(c, c % 2)
        pending = nxt

    pltpu.sync_copy(out_v, out_hbm.at[pl.ds(wid * _ROWS_PER_W,
                                            _ROWS_PER_W)])


@jax.jit
def _fm_call(idx_flat, table, wb):
    mesh = plsc.VectorSubcoreMesh(core_axis_name="c", subcore_axis_name="s")
    return pl.kernel(
        _fm_body,
        mesh=mesh,
        out_type=jax.ShapeDtypeStruct((_B,), jnp.float32),
        compiler_params=pltpu.CompilerParams(needs_layout_passes=False,
                                             use_tc_tiling_on_sc=False),
        scratch_types=[
            pltpu.VMEM((_ROWS_PER_W * _F,), jnp.int32),
            pltpu.VMEM((2, _IDX_PER_CHUNK, _D), jnp.float32),
            pltpu.VMEM((_ROWS_PER_W,), jnp.float32),
            pltpu.VMEM((48,), jnp.float32),
            pltpu.SemaphoreType.DMA,
            pltpu.SemaphoreType.DMA,
        ],
    )(idx_flat, table, wb)


def kernel(interaction_pairs, emb_table, lin_w, lin_b):
    t_lin = _tr_call(emb_table.T)
    table_c = t_lin.reshape(_VOCAB, _D)
    idx_flat = interaction_pairs.reshape(-1).astype(jnp.int32)
    w = lin_w.reshape(-1)
    w0 = w[:16]
    w1 = jnp.where(jnp.arange(16) >= 6,
                   jnp.concatenate([jnp.zeros((6,), w.dtype), w[16:26]]),
                   0.0).astype(jnp.float32)
    wb = jnp.concatenate([w0, w1,
                          jnp.full((16,), lin_b.reshape(-1)[0] / 16.0,
                                   jnp.float32)])
    return _fm_call(idx_flat, table_c, wb)


# speedup vs baseline: 1.0436x; 1.0436x over previous
import functools

import jax
import jax.numpy as jnp
from jax import lax
from jax.experimental import pallas as pl
from jax.experimental.pallas import tpu as pltpu
from jax.experimental.pallas import tpu_sc as plsc

_VOCAB = 1000000
_D = 16
_F = 26
_B = 16384

_SLAB = 512
_NSLAB_FULL = _VOCAB // _SLAB
_TAIL = _VOCAB - _NSLAB_FULL * _SLAB
_TRIPLES = 21

_NC = 2
_NS = 16
_NW = _NC * _NS
_ROWS_PER_W = _B // _NW
_CHUNK = 64
_NCHUNK = _ROWS_PER_W // _CHUNK
_IDX_PER_CHUNK = _CHUNK * _F
_GATHER = 416
_GPC = _IDX_PER_CHUNK // _GATHER
_IDXROWS_PER_W = _ROWS_PER_W * _F // _GATHER


def _tr_body(tab_t_hbm, out_hbm, slabs, out_bufs, sem_in, sem_out):
    wid = lax.axis_index("s") * _NC + lax.axis_index("c")
    nper = jnp.where(wid == 0, 62, 61)
    base = 61 * wid + jnp.minimum(wid, 1)

    lane = lax.iota(jnp.int32, 16)
    zq4 = lax.shift_right_logical(lane, 3) * 4
    yv = lax.bitwise_and(lane, 7)
    bufv = tuple(jnp.full((16,), b, jnp.int32) for b in range(3))

    def issue_in(s, buf):
        col0 = s * _SLAB
        for i in range(2):
            for jj in range(4):
                pltpu.async_copy(
                    tab_t_hbm.at[pl.ds(8 * i, 8),
                                 pl.ds(col0 + 128 * jj, 128)],
                    slabs.at[buf, i * 4 + jj], sem_in)

    def drain_in(buf):
        for t in range(8):
            pltpu.make_async_copy(
                tab_t_hbm.at[pl.ds(0, 8), pl.ds(0, 128)],
                slabs.at[buf, t], sem_in).wait()

    def drain_out(buf):
        pltpu.make_async_copy(out_hbm.at[pl.ds(0, _SLAB * 16)],
                              out_bufs.at[buf], sem_out).wait()

    diags = [lax.bitwise_and(lane + sh, 15) for sh in range(16)]
    sbases = [diags[sh] * 16 + lane for sh in range(16)]
    zero16 = jnp.zeros((16,), jnp.int32)
    gbase0 = zq4 * 1024 + yv * 128

    def compute(s, buf):
        col0 = s * _SLAB

        def tile_body(t, carry):
            gbt = gbase0 + (t * 1024 + buf * 8192)
            obase = t * 2048

            def g_body(g, carry2):
                cc0 = g * 16
                ga = gbt + cc0
                sb = lane + (obase + cc0 * 16)
                for sh in range(16):
                    v = plsc.load_gather(
                        slabs, [zero16, zero16, zero16, ga + diags[sh]])
                    plsc.store_scatter(
                        out_bufs, [bufv[buf], sb + sbases[sh]], v)
                return carry2

            lax.fori_loop(0, 8, g_body, carry)
            return carry

        lax.fori_loop(0, 4, tile_body, jnp.int32(0))
        pltpu.async_copy(out_bufs.at[buf],
                         out_hbm.at[pl.ds(col0 * 16, _SLAB * 16)], sem_out)

    issue_in(base, 0)
    issue_in(base + 1, 1)

    def slot(k, buf):
        @pl.when(k < nper)
        def _():
            drain_in(buf)

            @pl.when(k + 2 < nper)
            def _():
                issue_in(base + k + 2, (buf + 2) % 3)

            @pl.when(k >= 3)
            def _():
                drain_out(buf)

            compute(base + k, buf)

    def triple_body(p, carry):
        k = 3 * p
        slot(k, 0)
        slot(k + 1, 1)
        slot(k + 2, 2)
        return carry

    lax.fori_loop(0, _TRIPLES, triple_body, jnp.int32(0))
    drain_out(0)
    drain_out(1)
    drain_out(2)


@jax.jit
def _tr_call(tab_t):
    mesh = plsc.VectorSubcoreMesh(core_axis_name="c", subcore_axis_name="s")
    return pl.kernel(
        _tr_body,
        mesh=mesh,
        out_type=jax.ShapeDtypeStruct((_VOCAB * _D,), jnp.float32),
        compiler_params=pltpu.CompilerParams(needs_layout_passes=False,
                                             use_tc_tiling_on_sc=True),
        scratch_types=[
            pltpu.VMEM((3, 8, 8, 128), jnp.float32),
            pltpu.VMEM((4, _SLAB * 16), jnp.float32),
            pltpu.SemaphoreType.DMA,
            pltpu.SemaphoreType.DMA,
        ],
    )(tab_t)


def _fm_body(idx_flat_hbm, table_hbm, wb_hbm, out_hbm,
             idx_lin, rows, out_v, wconst, sem0, sem1):
    wid = lax.axis_index("s") * _NC + lax.axis_index("c")
    sems = (sem0, sem1)

    pltpu.sync_copy(idx_flat_hbm.at[pl.ds(wid * (_ROWS_PER_W * _F),
                                          _ROWS_PER_W * _F)], idx_lin)
    pltpu.sync_copy(wb_hbm, wconst)

    w0 = wconst[pl.ds(0, 16)]
    w1 = wconst[pl.ds(16, 16)]
    bias16 = wconst[pl.ds(32, 16)]

    def issue(c, buf):
        handles = []
        for j in range(_GPC):
            g = c * _GPC + j
            handles.append(pltpu.async_copy(
                table_hbm.at[idx_lin.at[pl.ds(g * _GATHER, _GATHER)]],
                rows.at[buf, pl.ds(j * _GATHER, _GATHER), :],
                sems[buf]))
        return handles

    def compute(c, buf):
        def row_body(lb, _):
            base = lb * _F
            boff = c * _IDX_PER_CHUNK + base
            v = rows[buf, base, :]
            s = v
            q = v * v
            for f in range(1, _F):
                v = rows[buf, base + f, :]
                s = s + v
                q = q + v * v
            i0 = idx_lin[pl.ds(boff, 16)].astype(jnp.float32)
            i1 = idx_lin[pl.ds(boff + 10, 16)].astype(jnp.float32)
            t = 0.5 * (s * s - q) + i0 * w0 + i1 * w1 + bias16
            dest = jnp.full((16,), c * _CHUNK + lb, jnp.int32)
            plsc.addupdate_scatter(out_v, [dest], t)
            return _

        lax.fori_loop(0, _CHUNK, row_body, jnp.int32(0))

    def zero_out(k, _):
        out_v[pl.ds(k * 16, 16)] = jnp.zeros((16,), jnp.float32)
        return _

    lax.fori_loop(0, _ROWS_PER_W // 16, zero_out, jnp.int32(0))

    pending = issue(0, 0)
    for c in range(_NCHUNK):
        nxt = issue(c + 1, (c + 1) % 2) if c + 1 < _NCHUNK else []
        for h in pending:
            h.wait()
        compute(c, c % 2)
        pending = nxt

    pltpu.sync_copy(out_v, out_hbm.at[pl.ds(wid * _ROWS_PER_W,
                                            _ROWS_PER_W)])


@jax.jit
def _fm_call(idx_flat, table, wb):
    mesh = plsc.VectorSubcoreMesh(core_axis_name="c", subcore_axis_name="s")
    return pl.kernel(
        _fm_body,
        mesh=mesh,
        out_type=jax.ShapeDtypeStruct((_B,), jnp.float32),
        compiler_params=pltpu.CompilerParams(needs_layout_passes=False,
                                             use_tc_tiling_on_sc=False),
        scratch_types=[
            pltpu.VMEM((_ROWS_PER_W * _F,), jnp.int32),
            pltpu.VMEM((2, _IDX_PER_CHUNK, _D), jnp.float32),
            pltpu.VMEM((_ROWS_PER_W,), jnp.float32),
            pltpu.VMEM((48,), jnp.float32),
            pltpu.SemaphoreType.DMA,
            pltpu.SemaphoreType.DMA,
        ],
    )(idx_flat, table, wb)


def kernel(interaction_pairs, emb_table, lin_w, lin_b):
    t_lin = _tr_call(emb_table.T)
    tail = lax.slice(emb_table, (_NSLAB_FULL * _SLAB, 0), (_VOCAB, _D))
    t_lin = lax.dynamic_update_slice(
        t_lin, tail.reshape(-1), (jnp.int32(_NSLAB_FULL * _SLAB * _D),))
    table_c = t_lin.reshape(_VOCAB, _D)
    idx_flat = interaction_pairs.reshape(-1).astype(jnp.int32)
    w = lin_w.reshape(-1)
    w0 = w[:16]
    w1 = jnp.where(jnp.arange(16) >= 6,
                   jnp.concatenate([jnp.zeros((6,), w.dtype), w[16:26]]),
                   0.0).astype(jnp.float32)
    wb = jnp.concatenate([w0, w1,
                          jnp.full((16,), lin_b.reshape(-1)[0] / 16.0,
                                   jnp.float32)])
    return _fm_call(idx_flat, table_c, wb)


# speedup vs baseline: 1.4165x; 1.3573x over previous
import functools

import jax
import jax.numpy as jnp
from jax import lax
from jax.experimental import pallas as pl
from jax.experimental.pallas import tpu as pltpu
from jax.experimental.pallas import tpu_sc as plsc

_VOCAB = 1000000
_D = 16
_F = 26
_B = 16384

_SLAB = 512
_NSLAB_FULL = _VOCAB // _SLAB
_TAIL = _VOCAB - _NSLAB_FULL * _SLAB
_TRIPLES = 21

_NC = 2
_NS = 16
_NW = _NC * _NS
_ROWS_PER_W = _B // _NW
_CHUNK = 64
_NCHUNK = _ROWS_PER_W // _CHUNK
_IDX_PER_CHUNK = _CHUNK * _F
_GATHER = 416
_GPC = _IDX_PER_CHUNK // _GATHER
_IDXROWS_PER_W = _ROWS_PER_W * _F // _GATHER


def _tr_body(tab_t_hbm, out_hbm, slabs, out_bufs, sem_in, sem_out):
    wid = lax.axis_index("s") * _NC + lax.axis_index("c")
    nper = jnp.where(wid == 0, 62, 61)
    base = 61 * wid + jnp.minimum(wid, 1)

    lane = lax.iota(jnp.int32, 16)
    zq4 = lax.shift_right_logical(lane, 3) * 4
    yv = lax.bitwise_and(lane, 7)
    bufv = tuple(jnp.full((16,), b, jnp.int32) for b in range(3))

    def issue_in(s, buf):
        col0 = s * _SLAB
        for i in range(2):
            for jj in range(4):
                pltpu.async_copy(
                    tab_t_hbm.at[pl.ds(8 * i, 8),
                                 pl.ds(col0 + 128 * jj, 128)],
                    slabs.at[buf, i * 4 + jj], sem_in)

    def drain_in(buf):
        for t in range(8):
            pltpu.make_async_copy(
                tab_t_hbm.at[pl.ds(0, 8), pl.ds(0, 128)],
                slabs.at[buf, t], sem_in).wait()

    def drain_out(buf):
        pltpu.make_async_copy(out_hbm.at[pl.ds(0, _SLAB * 16)],
                              out_bufs.at[buf], sem_out).wait()

    diags = [lax.bitwise_and(lane + sh, 15) for sh in range(16)]
    sbases = [diags[sh] * 16 + lane for sh in range(16)]
    zero16 = jnp.zeros((16,), jnp.int32)
    gbase0 = zq4 * 1024 + yv * 128

    def compute(s, buf):
        col0 = s * _SLAB

        def tile_body(t, carry):
            gbt = gbase0 + (t * 1024 + buf * 8192)
            obase = t * 2048

            @plsc.parallel_loop(0, 8, unroll=2)
            def g_body(g):
                cc0 = g * 16
                ga = gbt + cc0
                sb = lane + (obase + cc0 * 16)
                for sh in range(16):
                    v = plsc.load_gather(
                        slabs, [zero16, zero16, zero16, ga + diags[sh]])
                    plsc.store_scatter(
                        out_bufs, [bufv[buf], sb + sbases[sh]], v)

            return carry

        lax.fori_loop(0, 4, tile_body, jnp.int32(0))
        pltpu.async_copy(out_bufs.at[buf],
                         out_hbm.at[pl.ds(col0 * 16, _SLAB * 16)], sem_out)

    issue_in(base, 0)
    issue_in(base + 1, 1)

    def slot(k, buf):
        @pl.when(k < nper)
        def _():
            drain_in(buf)

            @pl.when(k + 2 < nper)
            def _():
                issue_in(base + k + 2, (buf + 2) % 3)

            @pl.when(k >= 3)
            def _():
                drain_out(buf)

            compute(base + k, buf)

    def triple_body(p, carry):
        k = 3 * p
        slot(k, 0)
        slot(k + 1, 1)
        slot(k + 2, 2)
        return carry

    lax.fori_loop(0, _TRIPLES, triple_body, jnp.int32(0))
    drain_out(0)
    drain_out(1)
    drain_out(2)


@jax.jit
def _tr_call(tab_t):
    mesh = plsc.VectorSubcoreMesh(core_axis_name="c", subcore_axis_name="s")
    return pl.kernel(
        _tr_body,
        mesh=mesh,
        out_type=jax.ShapeDtypeStruct((_VOCAB * _D,), jnp.float32),
        compiler_params=pltpu.CompilerParams(needs_layout_passes=False,
                                             use_tc_tiling_on_sc=True),
        scratch_types=[
            pltpu.VMEM((3, 8, 8, 128), jnp.float32),
            pltpu.VMEM((4, _SLAB * 16), jnp.float32),
            pltpu.SemaphoreType.DMA,
            pltpu.SemaphoreType.DMA,
        ],
    )(tab_t)


def _fm_body(idx_flat_hbm, table_hbm, wb_hbm, out_hbm,
             idx_lin, rows, out_v, wconst, sem0, sem1):
    wid = lax.axis_index("s") * _NC + lax.axis_index("c")
    sems = (sem0, sem1)

    pltpu.sync_copy(idx_flat_hbm.at[pl.ds(wid * (_ROWS_PER_W * _F),
                                          _ROWS_PER_W * _F)], idx_lin)
    pltpu.sync_copy(wb_hbm, wconst)

    w0 = wconst[pl.ds(0, 16)]
    w1 = wconst[pl.ds(16, 16)]
    bias16 = wconst[pl.ds(32, 16)]

    def issue(c, buf):
        handles = []
        for j in range(_GPC):
            g = c * _GPC + j
            handles.append(pltpu.async_copy(
                table_hbm.at[idx_lin.at[pl.ds(g * _GATHER, _GATHER)]],
                rows.at[buf, pl.ds(j * _GATHER, _GATHER), :],
                sems[buf]))
        return handles

    def compute(c, buf):
        def row_body(lb, _):
            base = lb * _F
            boff = c * _IDX_PER_CHUNK + base
            v = rows[buf, base, :]
            s = v
            q = v * v
            for f in range(1, _F):
                v = rows[buf, base + f, :]
                s = s + v
                q = q + v * v
            i0 = idx_lin[pl.ds(boff, 16)].astype(jnp.float32)
            i1 = idx_lin[pl.ds(boff + 10, 16)].astype(jnp.float32)
            t = 0.5 * (s * s - q) + i0 * w0 + i1 * w1 + bias16
            dest = jnp.full((16,), c * _CHUNK + lb, jnp.int32)
            plsc.addupdate_scatter(out_v, [dest], t)
            return _

        lax.fori_loop(0, _CHUNK, row_body, jnp.int32(0))

    def zero_out(k, _):
        out_v[pl.ds(k * 16, 16)] = jnp.zeros((16,), jnp.float32)
        return _

    lax.fori_loop(0, _ROWS_PER_W // 16, zero_out, jnp.int32(0))

    pending = issue(0, 0)
    for c in range(_NCHUNK):
        nxt = issue(c + 1, (c + 1) % 2) if c + 1 < _NCHUNK else []
        for h in pending:
            h.wait()
        compute(c, c % 2)
        pending = nxt

    pltpu.sync_copy(out_v, out_hbm.at[pl.ds(wid * _ROWS_PER_W,
                                            _ROWS_PER_W)])


@jax.jit
def _fm_call(idx_flat, table, wb):
    mesh = plsc.VectorSubcoreMesh(core_axis_name="c", subcore_axis_name="s")
    return pl.kernel(
        _fm_body,
        mesh=mesh,
        out_type=jax.ShapeDtypeStruct((_B,), jnp.float32),
        compiler_params=pltpu.CompilerParams(needs_layout_passes=False,
                                             use_tc_tiling_on_sc=False),
        scratch_types=[
            pltpu.VMEM((_ROWS_PER_W * _F,), jnp.int32),
            pltpu.VMEM((2, _IDX_PER_CHUNK, _D), jnp.float32),
            pltpu.VMEM((_ROWS_PER_W,), jnp.float32),
            pltpu.VMEM((48,), jnp.float32),
            pltpu.SemaphoreType.DMA,
            pltpu.SemaphoreType.DMA,
        ],
    )(idx_flat, table, wb)


def kernel(interaction_pairs, emb_table, lin_w, lin_b):
    t_lin = _tr_call(emb_table.T)
    tail = lax.slice(emb_table, (_NSLAB_FULL * _SLAB, 0), (_VOCAB, _D))
    t_lin = lax.dynamic_update_slice(
        t_lin, tail.reshape(-1), (jnp.int32(_NSLAB_FULL * _SLAB * _D),))
    table_c = t_lin.reshape(_VOCAB, _D)
    idx_flat = interaction_pairs.reshape(-1).astype(jnp.int32)
    w = lin_w.reshape(-1)
    w0 = w[:16]
    w1 = jnp.where(jnp.arange(16) >= 6,
                   jnp.concatenate([jnp.zeros((6,), w.dtype), w[16:26]]),
                   0.0).astype(jnp.float32)
    wb = jnp.concatenate([w0, w1,
                          jnp.full((16,), lin_b.reshape(-1)[0] / 16.0,
                                   jnp.float32)])
    return _fm_call(idx_flat, table_c, wb)


# speedup vs baseline: 1.6549x; 1.1683x over previous
import functools

import jax
import jax.numpy as jnp
from jax import lax
from jax.experimental import pallas as pl
from jax.experimental.pallas import tpu as pltpu
from jax.experimental.pallas import tpu_sc as plsc

_VOCAB = 1000000
_D = 16
_F = 26
_B = 16384

_SLAB = 512
_NSLAB_FULL = _VOCAB // _SLAB
_TAIL = _VOCAB - _NSLAB_FULL * _SLAB
_TRIPLES = 21

_NC = 2
_NS = 16
_NW = _NC * _NS
_ROWS_PER_W = _B // _NW
_CHUNK = 64
_NCHUNK = _ROWS_PER_W // _CHUNK
_IDX_PER_CHUNK = _CHUNK * _F
_GATHER = 416
_GPC = _IDX_PER_CHUNK // _GATHER
_IDXROWS_PER_W = _ROWS_PER_W * _F // _GATHER


def _tr_body(tab_t_hbm, out_hbm, slabs, out_bufs, sem_in, sem_out):
    wid = lax.axis_index("s") * _NC + lax.axis_index("c")
    nper = jnp.where(wid == 0, 62, 61)
    base = 61 * wid + jnp.minimum(wid, 1)

    lane = lax.iota(jnp.int32, 16)
    zq4 = lax.shift_right_logical(lane, 3) * 4
    yv = lax.bitwise_and(lane, 7)
    bufv = tuple(jnp.full((16,), b, jnp.int32) for b in range(3))

    def issue_in(s, buf):
        col0 = s * _SLAB
        for i in range(2):
            for jj in range(4):
                pltpu.async_copy(
                    tab_t_hbm.at[pl.ds(8 * i, 8),
                                 pl.ds(col0 + 128 * jj, 128)],
                    slabs.at[buf, i * 4 + jj], sem_in)

    def drain_in(buf):
        for t in range(8):
            pltpu.make_async_copy(
                tab_t_hbm.at[pl.ds(0, 8), pl.ds(0, 128)],
                slabs.at[buf, t], sem_in).wait()

    def drain_out(buf):
        pltpu.make_async_copy(out_hbm.at[pl.ds(0, _SLAB * 16)],
                              out_bufs.at[buf], sem_out).wait()

    diags = [lax.bitwise_and(lane + sh, 15) for sh in range(16)]
    sbases = [diags[sh] * 16 + lane for sh in range(16)]
    zero16 = jnp.zeros((16,), jnp.int32)
    gbase0 = zq4 * 1024 + yv * 128

    def compute(s, buf):
        col0 = s * _SLAB

        def tile_body(t, carry):
            gbt = gbase0 + (t * 1024 + buf * 8192)
            obase = t * 2048

            @plsc.parallel_loop(0, 8, unroll=4)
            def g_body(g):
                cc0 = g * 16
                ga = gbt + cc0
                sb = lane + (obase + cc0 * 16)
                for sh in range(16):
                    v = plsc.load_gather(
                        slabs, [zero16, zero16, zero16, ga + diags[sh]])
                    plsc.store_scatter(
                        out_bufs, [bufv[buf], sb + sbases[sh]], v)

            return carry

        lax.fori_loop(0, 4, tile_body, jnp.int32(0))
        pltpu.async_copy(out_bufs.at[buf],
                         out_hbm.at[pl.ds(col0 * 16, _SLAB * 16)], sem_out)

    issue_in(base, 0)
    issue_in(base + 1, 1)

    def slot(k, buf):
        @pl.when(k < nper)
        def _():
            drain_in(buf)

            @pl.when(k + 2 < nper)
            def _():
                issue_in(base + k + 2, (buf + 2) % 3)

            @pl.when(k >= 3)
            def _():
                drain_out(buf)

            compute(base + k, buf)

    def triple_body(p, carry):
        k = 3 * p
        slot(k, 0)
        slot(k + 1, 1)
        slot(k + 2, 2)
        return carry

    lax.fori_loop(0, _TRIPLES, triple_body, jnp.int32(0))
    drain_out(0)
    drain_out(1)
    drain_out(2)


@jax.jit
def _tr_call(tab_t):
    mesh = plsc.VectorSubcoreMesh(core_axis_name="c", subcore_axis_name="s")
    return pl.kernel(
        _tr_body,
        mesh=mesh,
        out_type=jax.ShapeDtypeStruct((_VOCAB * _D,), jnp.float32),
        compiler_params=pltpu.CompilerParams(needs_layout_passes=False,
                                             use_tc_tiling_on_sc=True),
        scratch_types=[
            pltpu.VMEM((3, 8, 8, 128), jnp.float32),
            pltpu.VMEM((4, _SLAB * 16), jnp.float32),
            pltpu.SemaphoreType.DMA,
            pltpu.SemaphoreType.DMA,
        ],
    )(tab_t)


def _fm_body(idx_flat_hbm, table_hbm, wb_hbm, out_hbm,
             idx_lin, rows, out_v, wconst, sem0, sem1):
    wid = lax.axis_index("s") * _NC + lax.axis_index("c")
    sems = (sem0, sem1)

    pltpu.sync_copy(idx_flat_hbm.at[pl.ds(wid * (_ROWS_PER_W * _F),
                                          _ROWS_PER_W * _F)], idx_lin)
    pltpu.sync_copy(wb_hbm, wconst)

    w0 = wconst[pl.ds(0, 16)]
    w1 = wconst[pl.ds(16, 16)]
    bias16 = wconst[pl.ds(32, 16)]

    def issue(c, buf):
        handles = []
        for j in range(_GPC):
            g = c * _GPC + j
            handles.append(pltpu.async_copy(
                table_hbm.at[idx_lin.at[pl.ds(g * _GATHER, _GATHER)]],
                rows.at[buf, pl.ds(j * _GATHER, _GATHER), :],
                sems[buf]))
        return handles

    def compute(c, buf):
        @plsc.parallel_loop(0, _CHUNK, unroll=2)
        def row_body(lb):
            base = lb * _F
            boff = c * _IDX_PER_CHUNK + base
            v = rows[buf, base, :]
            s = v
            q = v * v
            for f in range(1, _F):
                v = rows[buf, base + f, :]
                s = s + v
                q = q + v * v
            i0 = idx_lin[pl.ds(boff, 16)].astype(jnp.float32)
            i1 = idx_lin[pl.ds(boff + 10, 16)].astype(jnp.float32)
            t = 0.5 * (s * s - q) + i0 * w0 + i1 * w1 + bias16
            dest = jnp.full((16,), c * _CHUNK + lb, jnp.int32)
            plsc.addupdate_scatter(out_v, [dest], t)

    def zero_out(k, _):
        out_v[pl.ds(k * 16, 16)] = jnp.zeros((16,), jnp.float32)
        return _

    lax.fori_loop(0, _ROWS_PER_W // 16, zero_out, jnp.int32(0))

    pending = issue(0, 0)
    for c in range(_NCHUNK):
        nxt = issue(c + 1, (c + 1) % 2) if c + 1 < _NCHUNK else []
        for h in pending:
            h.wait()
        compute(c, c % 2)
        pending = nxt

    pltpu.sync_copy(out_v, out_hbm.at[pl.ds(wid * _ROWS_PER_W,
                                            _ROWS_PER_W)])


@jax.jit
def _fm_call(idx_flat, table, wb):
    mesh = plsc.VectorSubcoreMesh(core_axis_name="c", subcore_axis_name="s")
    return pl.kernel(
        _fm_body,
        mesh=mesh,
        out_type=jax.ShapeDtypeStruct((_B,), jnp.float32),
        compiler_params=pltpu.CompilerParams(needs_layout_passes=False,
                                             use_tc_tiling_on_sc=False),
        scratch_types=[
            pltpu.VMEM((_ROWS_PER_W * _F,), jnp.int32),
            pltpu.VMEM((2, _IDX_PER_CHUNK, _D), jnp.float32),
            pltpu.VMEM((_ROWS_PER_W,), jnp.float32),
            pltpu.VMEM((48,), jnp.float32),
            pltpu.SemaphoreType.DMA,
            pltpu.SemaphoreType.DMA,
        ],
    )(idx_flat, table, wb)


def kernel(interaction_pairs, emb_table, lin_w, lin_b):
    t_lin = _tr_call(emb_table.T)
    tail = lax.slice(emb_table, (_NSLAB_FULL * _SLAB, 0), (_VOCAB, _D))
    t_lin = lax.dynamic_update_slice(
        t_lin, tail.reshape(-1), (jnp.int32(_NSLAB_FULL * _SLAB * _D),))
    table_c = t_lin.reshape(_VOCAB, _D)
    idx_flat = interaction_pairs.reshape(-1).astype(jnp.int32)
    w = lin_w.reshape(-1)
    w0 = w[:16]
    w1 = jnp.where(jnp.arange(16) >= 6,
                   jnp.concatenate([jnp.zeros((6,), w.dtype), w[16:26]]),
                   0.0).astype(jnp.float32)
    wb = jnp.concatenate([w0, w1,
                          jnp.full((16,), lin_b.reshape(-1)[0] / 16.0,
                                   jnp.float32)])
    return _fm_call(idx_flat, table_c, wb)
